# baseline (speedup 1.0000x reference)
import functools

import jax
import jax.numpy as jnp
from jax import lax
from jax.experimental import pallas as pl
from jax.experimental.pallas import tpu as pltpu
from jax.experimental.pallas import tpu_sc as plsc

_D = 64
_L = 128
_NBLK = 2560


@functools.partial(jax.jit, static_argnames=("t_dim", "b_dim"))
def _gather_sc(idx2d, w2, t_dim, b_dim):
    info = plsc.get_sparse_core_info()
    nw = info.num_cores * info.num_subcores
    blk_per_w = _NBLK // nw
    pairs = blk_per_w // 2
    mesh = plsc.VectorSubcoreMesh(core_axis_name="c", subcore_axis_name="s")

    @functools.partial(
        pl.kernel,
        mesh=mesh,
        out_type=jax.ShapeDtypeStruct((t_dim, _D, b_dim), jnp.float32),
        scratch_types=[
            pltpu.VMEM((blk_per_w, _L), jnp.int32),
            pltpu.VMEM((blk_per_w, _L), jnp.int32),
            pltpu.VMEM((2, _L, _L), jnp.float32),
            pltpu.VMEM((2, _D, _L), jnp.float32),
            pltpu.SemaphoreType.DMA,
            pltpu.SemaphoreType.DMA,
            pltpu.SemaphoreType.DMA,
            pltpu.SemaphoreType.DMA,
        ],
        compiler_params=pltpu.CompilerParams(needs_layout_passes=False),
    )
    def k(idx_hbm, w_hbm, out_hbm, idx_v, hi_v, raw_v, out_v,
          g0, g1, w0, w1):
        wid = lax.axis_index("s") * info.num_cores + lax.axis_index("c")
        r0 = wid * blk_per_w
        gsems = (g0, g1)
        wsems = (w0, w1)

        pltpu.sync_copy(idx_hbm.at[pl.ds(r0, blk_per_w)], idx_v)

        def mk_hi(r, carry):
            for g in range(_L // 16):
                v = idx_v[r, pl.ds(g * 16, 16)]
                hi_v[r, pl.ds(g * 16, 16)] = lax.shift_right_logical(v, 1)
            return carry
        lax.fori_loop(0, blk_per_w, mk_hi, 0)

        def desc_g(rloc, slot):
            return pltpu.make_async_copy(
                w_hbm.at[hi_v.at[rloc]], raw_v.at[slot], gsems[slot])

        def desc_w(rloc, slot):
            r = r0 + rloc
            t = r // (b_dim // _L)
            j0 = (r % (b_dim // _L)) * _L
            return pltpu.make_async_copy(
                out_v.at[slot], out_hbm.at[t, :, pl.ds(j0, _L)], wsems[slot])

        rows = [
            jnp.full((16,), g * 16, jnp.int32) + lax.iota(jnp.int32, 16)
            for g in range(_L // 16)
        ]

        def transpose_block(rloc, slot):
            cbs = []
            for g in range(_L // 16):
                v = idx_v[rloc, pl.ds(g * 16, 16)]
                cbs.append(lax.shift_left(v & 1, 6))

            def dbody(d, cb):
                for g in range(_L // 16):
                    val = plsc.load_gather(
                        raw_v.at[slot], [rows[g], cb[g] + d])
                    out_v[slot, d, pl.ds(g * 16, 16)] = val
                return cb
            lax.fori_loop(0, _D, dbody, tuple(cbs))

        desc_g(0, 0).start()

        def body(g, carry):
            for slot in (0, 1):
                rloc = 2 * g + slot
                other = 1 - slot
                desc_g(rloc, slot).wait()
                if slot == 0:
                    desc_g(rloc + 1, other).start()
                else:
                    @pl.when(g < pairs - 1)
                    def _():
                        desc_g(rloc + 1, other).start()

                @pl.when(g > 0)
                def _():
                    desc_w(rloc - 2, slot).wait()
                transpose_block(rloc, slot)
                desc_w(rloc, slot).start()
            return carry

        lax.fori_loop(0, pairs, body, 0)
        desc_w(blk_per_w - 2, 0).wait()
        desc_w(blk_per_w - 1, 1).wait()

    return k(idx2d, w2)


def kernel(input_, weight):
    b, t = input_.shape
    idx2d = input_.T.astype(jnp.int32).reshape(_NBLK, _L)
    w2 = weight.reshape(weight.shape[0] // 2, 2 * weight.shape[1])
    out = _gather_sc(idx2d, w2, t, b)
    return jnp.transpose(out, (2, 0, 1))


# speedup vs baseline: 1.6195x; 1.6195x over previous
import functools

import jax
import jax.numpy as jnp
from jax import lax
from jax.experimental import pallas as pl
from jax.experimental.pallas import tpu as pltpu
from jax.experimental.pallas import tpu_sc as plsc

_D = 64
_L = 128
_NBLK = 2560


@functools.partial(jax.jit, static_argnames=("t_dim", "b_dim"))
def _gather_sc(idx2d, w2, t_dim, b_dim):
    info = plsc.get_sparse_core_info()
    nw = info.num_cores * info.num_subcores
    blk_per_w = _NBLK // nw
    pairs = blk_per_w // 2
    mesh = plsc.VectorSubcoreMesh(core_axis_name="c", subcore_axis_name="s")

    @functools.partial(
        pl.kernel,
        mesh=mesh,
        out_type=jax.ShapeDtypeStruct((t_dim, _D, b_dim), jnp.float32),
        scratch_types=[
            pltpu.VMEM((blk_per_w, _L), jnp.int32),
            pltpu.VMEM((2, _L, _L), jnp.float32),
            pltpu.VMEM((2, _D, _L), jnp.float32),
            pltpu.SemaphoreType.DMA,
            pltpu.SemaphoreType.DMA,
            pltpu.SemaphoreType.DMA,
            pltpu.SemaphoreType.DMA,
        ],
        compiler_params=pltpu.CompilerParams(needs_layout_passes=False),
    )
    def k(idx_hbm, w_hbm, out_hbm, idx_v, raw_v, out_v, g0, g1, w0, w1):
        wid = lax.axis_index("s") * info.num_cores + lax.axis_index("c")
        r0 = wid * blk_per_w
        gsems = (g0, g1)
        wsems = (w0, w1)

        pltpu.sync_copy(idx_hbm.at[pl.ds(r0, blk_per_w)], idx_v)

        def desc_g(rloc, slot):
            return pltpu.make_async_copy(
                w_hbm.at[idx_v.at[rloc]], raw_v.at[slot], gsems[slot])

        def desc_w(rloc, slot):
            r = r0 + rloc
            t = r // (b_dim // _L)
            j0 = (r % (b_dim // _L)) * _L
            return pltpu.make_async_copy(
                out_v.at[slot], out_hbm.at[t, :, pl.ds(j0, _L)], wsems[slot])

        rows = [
            jnp.full((16,), g * 16, jnp.int32) + lax.iota(jnp.int32, 16)
            for g in range(_L // 16)
        ]

        def transpose_block(slot):
            @functools.partial(plsc.parallel_loop, 0, _D, unroll=8)
            def _(d):
                col = jnp.full((16,), 0, jnp.int32) + d
                for g in range(_L // 16):
                    val = plsc.load_gather(raw_v.at[slot], [rows[g], col])
                    out_v[slot, d, pl.ds(g * 16, 16)] = val

        desc_g(0, 0).start()

        def body(g, carry):
            for slot in (0, 1):
                rloc = 2 * g + slot
                other = 1 - slot
                desc_g(rloc, slot).wait()
                if slot == 0:
                    desc_g(rloc + 1, other).start()
                else:
                    @pl.when(g < pairs - 1)
                    def _():
                        desc_g(rloc + 1, other).start()

                @pl.when(g > 0)
                def _():
                    desc_w(rloc - 2, slot).wait()
                transpose_block(slot)
                desc_w(rloc, slot).start()
            return carry

        lax.fori_loop(0, pairs, body, 0)
        desc_w(blk_per_w - 2, 0).wait()
        desc_w(blk_per_w - 1, 1).wait()

    return k(idx2d, w2)


def kernel(input_, weight):
    b, t = input_.shape
    idx2d = input_.T.astype(jnp.int32).reshape(_NBLK, _L)
    w2 = jnp.pad(weight, ((0, 0), (0, 2 * _D - weight.shape[1])))
    out = _gather_sc(idx2d, w2, t, b)
    return jnp.transpose(out, (2, 0, 1))
